# Initial kernel scaffold; baseline (speedup 1.0000x reference)
import jax
import jax.numpy as jnp
from jax.experimental import pallas as pl


def kernel(x, edge_index, batch, W1l, b1, W1r, W2l, b2, W2r, Wlin, blin):
    raise NotImplementedError("write your pallas kernel here")


# speedup vs baseline: 5.6300x; 5.6300x over previous
import functools

import jax
import jax.numpy as jnp
from jax import lax
from jax.experimental import pallas as pl
from jax.experimental.pallas import tpu as pltpu
from jax.experimental.pallas import tpu_sc as plsc

NC = 2
NS = 16
K = 80


def _sc_segment_sum(table, src, dst, zeros_acc, with_counts):
    n, d = table.shape
    e = src.shape[0]
    nw = NC * NS
    assert e % (nw * K) == 0
    e_per_w = e // nw
    t_chunks = e_per_w // K
    row_step = (n // NS) // 8 * 8
    row_span = n - row_step * (NS - 1)
    assert row_span % 8 == 0 and row_span >= row_step

    mesh = plsc.VectorSubcoreMesh(core_axis_name="c", subcore_axis_name="s")

    out_type = [jax.ShapeDtypeStruct((NC, n, d), jnp.float32)]
    scratch = [
        pltpu.VMEM((K,), jnp.int32),
        pltpu.VMEM((K,), jnp.int32),
        pltpu.VMEM((K, d), jnp.float32),
        pltpu.VMEM_SHARED((n, d), jnp.float32),
        pltpu.SemaphoreType.DMA,
    ]
    if with_counts:
        out_type.append(jax.ShapeDtypeStruct((NC * n,), jnp.float32))
        scratch.append(pltpu.VMEM((K,), jnp.float32))
        scratch.append(pltpu.VMEM_SHARED((n,), jnp.float32))
        scratch.append(pltpu.VMEM((K,), jnp.float32))

    def body(table_h, src_h, dst_h, zacc_h, out_h, *rest):
        cntrow_v = None
        if with_counts:
            (cnt_out_h, src_v, dst_v, rows_v, acc_s, sem, ones_v, cnt_s,
             cntrow_v) = rest
        else:
            src_v, dst_v, rows_v, acc_s, sem = rest
        c = lax.axis_index("c")
        s = lax.axis_index("s")
        wid = s * NC + c
        base_e = wid * e_per_w
        r0 = s * row_step
        n_sub = row_span // K

        pltpu.sync_copy(zacc_h.at[pl.ds(0, K)], rows_v)
        if with_counts:
            for i in range(K // 16):
                ones_v[pl.ds(i * 16, 16)] = jnp.full((16,), 1.0, jnp.float32)
                cntrow_v[pl.ds(i * 16, 16)] = jnp.zeros((16,), jnp.float32)

        @pl.loop(0, n_sub)
        def _(j):
            pltpu.sync_copy(rows_v, acc_s.at[pl.ds(r0 + j * K, K)])

        if with_counts:
            @pl.loop(0, n_sub)
            def _(j):
                pltpu.sync_copy(cntrow_v, cnt_s.at[pl.ds(r0 + j * K, K)])

        plsc.subcore_barrier()

        @pl.loop(0, t_chunks)
        def _(t):
            e0 = base_e + t * K
            pltpu.sync_copy(src_h.at[pl.ds(e0, K)], src_v)
            pltpu.sync_copy(dst_h.at[pl.ds(e0, K)], dst_v)
            pltpu.async_copy(table_h.at[src_v], rows_v, sem).wait()
            pltpu.sync_copy(rows_v, acc_s.at[dst_v], add=True)
            if with_counts:
                pltpu.sync_copy(ones_v, cnt_s.at[dst_v], add=True)

        plsc.subcore_barrier()

        @pl.loop(0, n_sub)
        def _(j):
            pltpu.sync_copy(acc_s.at[pl.ds(r0 + j * K, K)], rows_v)
            pltpu.sync_copy(rows_v, out_h.at[c, pl.ds(r0 + j * K, K)])

        if with_counts:
            @pl.loop(0, n_sub)
            def _(j):
                pltpu.sync_copy(cnt_s.at[pl.ds(r0 + j * K, K)], cntrow_v)
                pltpu.sync_copy(cntrow_v,
                                cnt_out_h.at[pl.ds(c * n + r0 + j * K, K)])

    fn = pl.kernel(body, out_type=tuple(out_type), mesh=mesh,
                   scratch_types=tuple(scratch))
    return fn(table, src, dst, zeros_acc)


def _tc_layer_body(sp_ref, cnt_ref, x_ref, wl_ref, b_ref, wr_ref, h_ref):
    ssum = sp_ref[0] + sp_ref[1]
    agg = ssum / jnp.maximum(cnt_ref[...], 1.0)
    t = (jnp.dot(agg, wl_ref[...], preferred_element_type=jnp.float32)
         + jnp.dot(x_ref[...], wr_ref[...], preferred_element_type=jnp.float32)
         + b_ref[...])
    nrm = jnp.sqrt(jnp.sum(t * t, axis=1, keepdims=True))
    t = t / jnp.maximum(nrm, 1e-12)
    h_ref[...] = jnp.maximum(t, 0.0)


def _tc_layer(sp, cnt, x, wl, b, wr, block_rows):
    n, d = x.shape
    grid = n // block_rows
    return pl.pallas_call(
        _tc_layer_body,
        grid=(grid,),
        in_specs=[
            pl.BlockSpec((NC, block_rows, d), lambda i: (0, i, 0)),
            pl.BlockSpec((block_rows, 1), lambda i: (i, 0)),
            pl.BlockSpec((block_rows, d), lambda i: (i, 0)),
            pl.BlockSpec((d, d), lambda i: (0, 0)),
            pl.BlockSpec((1, d), lambda i: (0, 0)),
            pl.BlockSpec((d, d), lambda i: (0, 0)),
        ],
        out_specs=pl.BlockSpec((block_rows, d), lambda i: (i, 0)),
        out_shape=jax.ShapeDtypeStruct((n, d), jnp.float32),
    )(sp, cnt, x, wl, b, wr)


def _tc_layer_pool_body(g, sp_ref, cnt_ref, x_ref, wl_ref, b_ref, wr_ref,
                        batch_ref, wlin_ref, blin_ref, out_ref,
                        gsum_ref, gcnt_ref):
    i = pl.program_id(0)
    ssum = sp_ref[0] + sp_ref[1]
    agg = ssum / jnp.maximum(cnt_ref[...], 1.0)
    t = (jnp.dot(agg, wl_ref[...], preferred_element_type=jnp.float32)
         + jnp.dot(x_ref[...], wr_ref[...], preferred_element_type=jnp.float32)
         + b_ref[...])
    nrm = jnp.sqrt(jnp.sum(t * t, axis=1, keepdims=True))
    h = jnp.maximum(t / jnp.maximum(nrm, 1e-12), 0.0)

    @pl.when(i == 0)
    def _():
        gsum_ref[...] = jnp.zeros_like(gsum_ref)
        gcnt_ref[...] = jnp.zeros_like(gcnt_ref)

    bids = batch_ref[0]
    gids = lax.broadcasted_iota(jnp.int32, (gsum_ref.shape[0], h.shape[0]), 0)
    m = (gids == bids).astype(jnp.float32)
    gsum_ref[...] += jnp.dot(m, h, preferred_element_type=jnp.float32)
    gcnt_ref[...] += jnp.sum(m, axis=1, keepdims=True)

    @pl.when(i == g - 1)
    def _():
        gmean = gsum_ref[...] / jnp.maximum(gcnt_ref[...], 1.0)
        out_ref[...] = (jnp.dot(gmean, wlin_ref[...],
                                preferred_element_type=jnp.float32)
                        + blin_ref[...])


def _tc_layer_pool(sp, cnt, x, wl, b, wr, batch3d, wlin, blin, g, block_rows):
    n, d = x.shape
    o = wlin.shape[1]
    grid = n // block_rows
    return pl.pallas_call(
        functools.partial(_tc_layer_pool_body, grid),
        grid=(grid,),
        in_specs=[
            pl.BlockSpec((NC, block_rows, d), lambda i: (0, i, 0)),
            pl.BlockSpec((block_rows, 1), lambda i: (i, 0)),
            pl.BlockSpec((block_rows, d), lambda i: (i, 0)),
            pl.BlockSpec((d, d), lambda i: (0, 0)),
            pl.BlockSpec((1, d), lambda i: (0, 0)),
            pl.BlockSpec((d, d), lambda i: (0, 0)),
            pl.BlockSpec((1, 1, block_rows), lambda i: (i, 0, 0)),
            pl.BlockSpec((d, o), lambda i: (0, 0)),
            pl.BlockSpec((1, o), lambda i: (0, 0)),
        ],
        out_specs=pl.BlockSpec((g, o), lambda i: (0, 0)),
        out_shape=jax.ShapeDtypeStruct((g, o), jnp.float32),
        scratch_shapes=[
            pltpu.VMEM((g, d), jnp.float32),
            pltpu.VMEM((g, 1), jnp.float32),
        ],
    )(sp, cnt, x, wl, b, wr, batch3d, wlin, blin)


BLOCK_ROWS = 1000


def kernel(x, edge_index, batch, W1l, b1, W1r, W2l, b2, W2r, Wlin, blin):
    n, d = x.shape
    src = edge_index[0]
    dst = edge_index[1]
    zeros_acc = jnp.zeros((n, d), jnp.float32)

    s1p, cntflat = _sc_segment_sum(x, src, dst, zeros_acc, with_counts=True)
    cnt = (cntflat[:n] + cntflat[n:]).reshape(n, 1)
    h1 = _tc_layer(s1p, cnt, x, W1l, b1.reshape(1, -1), W1r, BLOCK_ROWS)
    (s2p,) = _sc_segment_sum(h1, src, dst, zeros_acc, with_counts=False)
    batch3d = batch.reshape(n // BLOCK_ROWS, 1, BLOCK_ROWS)
    out = _tc_layer_pool(s2p, cnt, h1, W2l, b2.reshape(1, -1), W2r,
                         batch3d, Wlin, blin.reshape(1, -1),
                         g=64, block_rows=BLOCK_ROWS)
    return out


# speedup vs baseline: 7.9169x; 1.4062x over previous
import functools

import jax
import jax.numpy as jnp
from jax import lax
from jax.experimental import pallas as pl
from jax.experimental.pallas import tpu as pltpu
from jax.experimental.pallas import tpu_sc as plsc

NC = 2
NS = 16
K = 80


def _sc_segment_sum(table, src, dst, zeros_acc, with_counts):
    n, d = table.shape
    e = src.shape[0]
    nw = NC * NS
    assert e % (nw * K) == 0
    e_per_w = e // nw
    t_chunks = e_per_w // K
    row_step = (n // NS) // 8 * 8
    row_span = n - row_step * (NS - 1)
    assert row_span % 8 == 0 and row_span >= row_step

    mesh = plsc.VectorSubcoreMesh(core_axis_name="c", subcore_axis_name="s")

    out_type = [jax.ShapeDtypeStruct((NC, n, d), jnp.float32)]
    scratch = [
        pltpu.VMEM((e_per_w,), jnp.int32),
        pltpu.VMEM((t_chunks, K), jnp.int32),
        pltpu.VMEM((K, d), jnp.float32),
        pltpu.VMEM_SHARED((n, d), jnp.float32),
        pltpu.SemaphoreType.DMA,
    ]
    if with_counts:
        out_type.append(jax.ShapeDtypeStruct((NC * n,), jnp.float32))
        scratch.append(pltpu.VMEM((K,), jnp.float32))
        scratch.append(pltpu.VMEM_SHARED((n,), jnp.float32))
        scratch.append(pltpu.VMEM((K,), jnp.float32))

    def body(table_h, src_h, dst_h, zacc_h, out_h, *rest):
        cntrow_v = None
        if with_counts:
            (cnt_out_h, src_v, dst_v, rows_v, acc_s, sem, ones_v, cnt_s,
             cntrow_v) = rest
        else:
            src_v, dst_v, rows_v, acc_s, sem = rest
        c = lax.axis_index("c")
        s = lax.axis_index("s")
        wid = s * NC + c
        base_e = wid * e_per_w
        r0 = s * row_step
        n_sub = row_span // K

        pltpu.sync_copy(zacc_h.at[pl.ds(0, K)], rows_v)
        if with_counts:
            for i in range(K // 16):
                ones_v[pl.ds(i * 16, 16)] = jnp.full((16,), 1.0, jnp.float32)
                cntrow_v[pl.ds(i * 16, 16)] = jnp.zeros((16,), jnp.float32)

        @pl.loop(0, n_sub)
        def _(j):
            pltpu.sync_copy(rows_v, acc_s.at[pl.ds(r0 + j * K, K)])

        if with_counts:
            @pl.loop(0, n_sub)
            def _(j):
                pltpu.sync_copy(cntrow_v, cnt_s.at[pl.ds(r0 + j * K, K)])

        pltpu.sync_copy(src_h.at[pl.ds(base_e, e_per_w)], src_v)
        pltpu.sync_copy(dst_h.at[wid], dst_v)

        plsc.subcore_barrier()

        @pl.loop(0, t_chunks)
        def _(t):
            pltpu.async_copy(table_h.at[src_v.at[pl.ds(t * K, K)]],
                             rows_v, sem).wait()
            pltpu.sync_copy(rows_v, acc_s.at[dst_v.at[t]], add=True)
            if with_counts:
                pltpu.sync_copy(ones_v, cnt_s.at[dst_v.at[t]], add=True)

        plsc.subcore_barrier()

        @pl.loop(0, n_sub)
        def _(j):
            pltpu.sync_copy(acc_s.at[pl.ds(r0 + j * K, K)], rows_v)
            pltpu.sync_copy(rows_v, out_h.at[c, pl.ds(r0 + j * K, K)])

        if with_counts:
            @pl.loop(0, n_sub)
            def _(j):
                pltpu.sync_copy(cnt_s.at[pl.ds(r0 + j * K, K)], cntrow_v)
                pltpu.sync_copy(cntrow_v,
                                cnt_out_h.at[pl.ds(c * n + r0 + j * K, K)])

    fn = pl.kernel(body, out_type=tuple(out_type), mesh=mesh,
                   scratch_types=tuple(scratch))
    dst3 = dst.reshape(nw, t_chunks, K)
    return fn(table, src, dst3, zeros_acc)


def _tc_layer_body(sp_ref, cnt_ref, x_ref, wl_ref, b_ref, wr_ref, h_ref):
    ssum = sp_ref[0] + sp_ref[1]
    agg = ssum / jnp.maximum(cnt_ref[...], 1.0)
    t = (jnp.dot(agg, wl_ref[...], preferred_element_type=jnp.float32)
         + jnp.dot(x_ref[...], wr_ref[...], preferred_element_type=jnp.float32)
         + b_ref[...])
    nrm = jnp.sqrt(jnp.sum(t * t, axis=1, keepdims=True))
    t = t / jnp.maximum(nrm, 1e-12)
    h_ref[...] = jnp.maximum(t, 0.0)


def _tc_layer(sp, cnt, x, wl, b, wr, block_rows):
    n, d = x.shape
    grid = n // block_rows
    return pl.pallas_call(
        _tc_layer_body,
        grid=(grid,),
        in_specs=[
            pl.BlockSpec((NC, block_rows, d), lambda i: (0, i, 0)),
            pl.BlockSpec((block_rows, 1), lambda i: (i, 0)),
            pl.BlockSpec((block_rows, d), lambda i: (i, 0)),
            pl.BlockSpec((d, d), lambda i: (0, 0)),
            pl.BlockSpec((1, d), lambda i: (0, 0)),
            pl.BlockSpec((d, d), lambda i: (0, 0)),
        ],
        out_specs=pl.BlockSpec((block_rows, d), lambda i: (i, 0)),
        out_shape=jax.ShapeDtypeStruct((n, d), jnp.float32),
    )(sp, cnt, x, wl, b, wr)


def _tc_layer_pool_body(g, sp_ref, cnt_ref, x_ref, wl_ref, b_ref, wr_ref,
                        batch_ref, wlin_ref, blin_ref, out_ref,
                        gsum_ref, gcnt_ref):
    i = pl.program_id(0)
    ssum = sp_ref[0] + sp_ref[1]
    agg = ssum / jnp.maximum(cnt_ref[...], 1.0)
    t = (jnp.dot(agg, wl_ref[...], preferred_element_type=jnp.float32)
         + jnp.dot(x_ref[...], wr_ref[...], preferred_element_type=jnp.float32)
         + b_ref[...])
    nrm = jnp.sqrt(jnp.sum(t * t, axis=1, keepdims=True))
    h = jnp.maximum(t / jnp.maximum(nrm, 1e-12), 0.0)

    @pl.when(i == 0)
    def _():
        gsum_ref[...] = jnp.zeros_like(gsum_ref)
        gcnt_ref[...] = jnp.zeros_like(gcnt_ref)

    bids = batch_ref[0]
    gids = lax.broadcasted_iota(jnp.int32, (gsum_ref.shape[0], h.shape[0]), 0)
    m = (gids == bids).astype(jnp.float32)
    gsum_ref[...] += jnp.dot(m, h, preferred_element_type=jnp.float32)
    gcnt_ref[...] += jnp.sum(m, axis=1, keepdims=True)

    @pl.when(i == g - 1)
    def _():
        gmean = gsum_ref[...] / jnp.maximum(gcnt_ref[...], 1.0)
        out_ref[...] = (jnp.dot(gmean, wlin_ref[...],
                                preferred_element_type=jnp.float32)
                        + blin_ref[...])


def _tc_layer_pool(sp, cnt, x, wl, b, wr, batch3d, wlin, blin, g, block_rows):
    n, d = x.shape
    o = wlin.shape[1]
    grid = n // block_rows
    return pl.pallas_call(
        functools.partial(_tc_layer_pool_body, grid),
        grid=(grid,),
        in_specs=[
            pl.BlockSpec((NC, block_rows, d), lambda i: (0, i, 0)),
            pl.BlockSpec((block_rows, 1), lambda i: (i, 0)),
            pl.BlockSpec((block_rows, d), lambda i: (i, 0)),
            pl.BlockSpec((d, d), lambda i: (0, 0)),
            pl.BlockSpec((1, d), lambda i: (0, 0)),
            pl.BlockSpec((d, d), lambda i: (0, 0)),
            pl.BlockSpec((1, 1, block_rows), lambda i: (i, 0, 0)),
            pl.BlockSpec((d, o), lambda i: (0, 0)),
            pl.BlockSpec((1, o), lambda i: (0, 0)),
        ],
        out_specs=pl.BlockSpec((g, o), lambda i: (0, 0)),
        out_shape=jax.ShapeDtypeStruct((g, o), jnp.float32),
        scratch_shapes=[
            pltpu.VMEM((g, d), jnp.float32),
            pltpu.VMEM((g, 1), jnp.float32),
        ],
    )(sp, cnt, x, wl, b, wr, batch3d, wlin, blin)


BLOCK_ROWS = 1000


def kernel(x, edge_index, batch, W1l, b1, W1r, W2l, b2, W2r, Wlin, blin):
    n, d = x.shape
    src = edge_index[0]
    dst = edge_index[1]
    zeros_acc = jnp.zeros((n, d), jnp.float32)

    s1p, cntflat = _sc_segment_sum(x, src, dst, zeros_acc, with_counts=True)
    cnt = (cntflat[:n] + cntflat[n:]).reshape(n, 1)
    h1 = _tc_layer(s1p, cnt, x, W1l, b1.reshape(1, -1), W1r, BLOCK_ROWS)
    (s2p,) = _sc_segment_sum(h1, src, dst, zeros_acc, with_counts=False)
    batch3d = batch.reshape(n // BLOCK_ROWS, 1, BLOCK_ROWS)
    out = _tc_layer_pool(s2p, cnt, h1, W2l, b2.reshape(1, -1), W2r,
                         batch3d, Wlin, blin.reshape(1, -1),
                         g=64, block_rows=BLOCK_ROWS)
    return out


# speedup vs baseline: 10.0472x; 1.2691x over previous
import functools

import jax
import jax.numpy as jnp
from jax import lax
from jax.experimental import pallas as pl
from jax.experimental.pallas import tpu as pltpu
from jax.experimental.pallas import tpu_sc as plsc

NC = 2
NS = 16
K = 80


def _sc_segment_sum(table, src, dst, zeros_acc, with_counts):
    n, d = table.shape
    e = src.shape[0]
    nw = NC * NS
    assert e % (nw * K) == 0
    e_per_w = e // nw
    t_chunks = e_per_w // K
    row_step = (n // NS) // 8 * 8
    row_span = n - row_step * (NS - 1)
    assert row_span % 8 == 0 and row_span >= row_step

    mesh = plsc.VectorSubcoreMesh(core_axis_name="c", subcore_axis_name="s")

    nbuf = 2
    out_type = [jax.ShapeDtypeStruct((NC, n, d), jnp.float32)]
    scratch = [
        pltpu.VMEM((e_per_w,), jnp.int32),
        pltpu.VMEM((t_chunks, K), jnp.int32),
        pltpu.VMEM_SHARED((n, d), jnp.float32),
    ]
    scratch += [pltpu.VMEM((K, d), jnp.float32) for _ in range(nbuf)]
    scratch += [pltpu.SemaphoreType.DMA for _ in range(2 * nbuf + 1)]
    if with_counts:
        out_type.append(jax.ShapeDtypeStruct((NC * n,), jnp.float32))
        scratch.append(pltpu.VMEM((K,), jnp.float32))
        scratch.append(pltpu.VMEM_SHARED((n,), jnp.float32))
        scratch.append(pltpu.VMEM((K,), jnp.float32))

    def body(table_h, src_h, dst_h, zacc_h, out_h, *rest):
        cntrow_v = None
        if with_counts:
            cnt_out_h = rest[0]
            rest = rest[1:]
        src_v, dst_v, acc_s = rest[0], rest[1], rest[2]
        rows = list(rest[3:3 + nbuf])
        gsem = list(rest[3 + nbuf:3 + 2 * nbuf])
        ssem = list(rest[3 + 2 * nbuf:3 + 3 * nbuf])
        csem = rest[3 + 3 * nbuf]
        rows_v = rows[0]
        if with_counts:
            ones_v, cnt_s, cntrow_v = rest[3 + 3 * nbuf + 1:]
        c = lax.axis_index("c")
        s = lax.axis_index("s")
        wid = s * NC + c
        base_e = wid * e_per_w
        r0 = s * row_step
        n_sub = row_span // K

        pltpu.sync_copy(zacc_h.at[pl.ds(0, K)], rows_v)
        if with_counts:
            for i in range(K // 16):
                ones_v[pl.ds(i * 16, 16)] = jnp.full((16,), 1.0, jnp.float32)
                cntrow_v[pl.ds(i * 16, 16)] = jnp.zeros((16,), jnp.float32)

        @pl.loop(0, n_sub)
        def _(j):
            pltpu.sync_copy(rows_v, acc_s.at[pl.ds(r0 + j * K, K)])

        if with_counts:
            @pl.loop(0, n_sub)
            def _(j):
                pltpu.sync_copy(cntrow_v, cnt_s.at[pl.ds(r0 + j * K, K)])

        pltpu.sync_copy(src_h.at[pl.ds(base_e, e_per_w)], src_v)
        pltpu.sync_copy(dst_h.at[wid], dst_v)

        plsc.subcore_barrier()

        def do_chunks(t0, nb):
            gd, sd, cd = [], [], []
            for b in range(nb):
                gd.append(pltpu.async_copy(
                    table_h.at[src_v.at[pl.ds((t0 + b) * K, K)]],
                    rows[b], gsem[b]))
            for b in range(nb):
                gd[b].wait()
                sd.append(pltpu.async_copy(
                    rows[b], acc_s.at[dst_v.at[t0 + b]], ssem[b], add=True))
                if with_counts:
                    cd.append(pltpu.async_copy(
                        ones_v, cnt_s.at[dst_v.at[t0 + b]], csem, add=True))
            for x_ in sd + cd:
                x_.wait()

        n_full = t_chunks // nbuf
        rem = t_chunks - n_full * nbuf

        @pl.loop(0, n_full)
        def _(tt):
            do_chunks(tt * nbuf, nbuf)

        if rem:
            do_chunks(n_full * nbuf, rem)

        plsc.subcore_barrier()

        @pl.loop(0, n_sub)
        def _(j):
            pltpu.sync_copy(acc_s.at[pl.ds(r0 + j * K, K)], rows_v)
            pltpu.sync_copy(rows_v, out_h.at[c, pl.ds(r0 + j * K, K)])

        if with_counts:
            @pl.loop(0, n_sub)
            def _(j):
                pltpu.sync_copy(cnt_s.at[pl.ds(r0 + j * K, K)], cntrow_v)
                pltpu.sync_copy(cntrow_v,
                                cnt_out_h.at[pl.ds(c * n + r0 + j * K, K)])

    fn = pl.kernel(body, out_type=tuple(out_type), mesh=mesh,
                   scratch_types=tuple(scratch))
    dst3 = dst.reshape(nw, t_chunks, K)
    return fn(table, src, dst3, zeros_acc)


def _tc_layer_body(sp_ref, cnt_ref, x_ref, wl_ref, b_ref, wr_ref, h_ref):
    ssum = sp_ref[0] + sp_ref[1]
    agg = ssum / jnp.maximum(cnt_ref[...], 1.0)
    t = (jnp.dot(agg, wl_ref[...], preferred_element_type=jnp.float32)
         + jnp.dot(x_ref[...], wr_ref[...], preferred_element_type=jnp.float32)
         + b_ref[...])
    nrm = jnp.sqrt(jnp.sum(t * t, axis=1, keepdims=True))
    t = t / jnp.maximum(nrm, 1e-12)
    h_ref[...] = jnp.maximum(t, 0.0)


def _tc_layer(sp, cnt, x, wl, b, wr, block_rows):
    n, d = x.shape
    grid = n // block_rows
    return pl.pallas_call(
        _tc_layer_body,
        grid=(grid,),
        in_specs=[
            pl.BlockSpec((NC, block_rows, d), lambda i: (0, i, 0)),
            pl.BlockSpec((block_rows, 1), lambda i: (i, 0)),
            pl.BlockSpec((block_rows, d), lambda i: (i, 0)),
            pl.BlockSpec((d, d), lambda i: (0, 0)),
            pl.BlockSpec((1, d), lambda i: (0, 0)),
            pl.BlockSpec((d, d), lambda i: (0, 0)),
        ],
        out_specs=pl.BlockSpec((block_rows, d), lambda i: (i, 0)),
        out_shape=jax.ShapeDtypeStruct((n, d), jnp.float32),
    )(sp, cnt, x, wl, b, wr)


def _tc_layer_pool_body(g, sp_ref, cnt_ref, x_ref, wl_ref, b_ref, wr_ref,
                        batch_ref, wlin_ref, blin_ref, out_ref,
                        gsum_ref, gcnt_ref):
    i = pl.program_id(0)
    ssum = sp_ref[0] + sp_ref[1]
    agg = ssum / jnp.maximum(cnt_ref[...], 1.0)
    t = (jnp.dot(agg, wl_ref[...], preferred_element_type=jnp.float32)
         + jnp.dot(x_ref[...], wr_ref[...], preferred_element_type=jnp.float32)
         + b_ref[...])
    nrm = jnp.sqrt(jnp.sum(t * t, axis=1, keepdims=True))
    h = jnp.maximum(t / jnp.maximum(nrm, 1e-12), 0.0)

    @pl.when(i == 0)
    def _():
        gsum_ref[...] = jnp.zeros_like(gsum_ref)
        gcnt_ref[...] = jnp.zeros_like(gcnt_ref)

    bids = batch_ref[0]
    gids = lax.broadcasted_iota(jnp.int32, (gsum_ref.shape[0], h.shape[0]), 0)
    m = (gids == bids).astype(jnp.float32)
    gsum_ref[...] += jnp.dot(m, h, preferred_element_type=jnp.float32)
    gcnt_ref[...] += jnp.sum(m, axis=1, keepdims=True)

    @pl.when(i == g - 1)
    def _():
        gmean = gsum_ref[...] / jnp.maximum(gcnt_ref[...], 1.0)
        out_ref[...] = (jnp.dot(gmean, wlin_ref[...],
                                preferred_element_type=jnp.float32)
                        + blin_ref[...])


def _tc_layer_pool(sp, cnt, x, wl, b, wr, batch3d, wlin, blin, g, block_rows):
    n, d = x.shape
    o = wlin.shape[1]
    grid = n // block_rows
    return pl.pallas_call(
        functools.partial(_tc_layer_pool_body, grid),
        grid=(grid,),
        in_specs=[
            pl.BlockSpec((NC, block_rows, d), lambda i: (0, i, 0)),
            pl.BlockSpec((block_rows, 1), lambda i: (i, 0)),
            pl.BlockSpec((block_rows, d), lambda i: (i, 0)),
            pl.BlockSpec((d, d), lambda i: (0, 0)),
            pl.BlockSpec((1, d), lambda i: (0, 0)),
            pl.BlockSpec((d, d), lambda i: (0, 0)),
            pl.BlockSpec((1, 1, block_rows), lambda i: (i, 0, 0)),
            pl.BlockSpec((d, o), lambda i: (0, 0)),
            pl.BlockSpec((1, o), lambda i: (0, 0)),
        ],
        out_specs=pl.BlockSpec((g, o), lambda i: (0, 0)),
        out_shape=jax.ShapeDtypeStruct((g, o), jnp.float32),
        scratch_shapes=[
            pltpu.VMEM((g, d), jnp.float32),
            pltpu.VMEM((g, 1), jnp.float32),
        ],
    )(sp, cnt, x, wl, b, wr, batch3d, wlin, blin)


BLOCK_ROWS = 1000


def kernel(x, edge_index, batch, W1l, b1, W1r, W2l, b2, W2r, Wlin, blin):
    n, d = x.shape
    src = edge_index[0]
    dst = edge_index[1]
    zeros_acc = jnp.zeros((n, d), jnp.float32)

    s1p, cntflat = _sc_segment_sum(x, src, dst, zeros_acc, with_counts=True)
    cnt = (cntflat[:n] + cntflat[n:]).reshape(n, 1)
    h1 = _tc_layer(s1p, cnt, x, W1l, b1.reshape(1, -1), W1r, BLOCK_ROWS)
    (s2p,) = _sc_segment_sum(h1, src, dst, zeros_acc, with_counts=False)
    batch3d = batch.reshape(n // BLOCK_ROWS, 1, BLOCK_ROWS)
    out = _tc_layer_pool(s2p, cnt, h1, W2l, b2.reshape(1, -1), W2r,
                         batch3d, Wlin, blin.reshape(1, -1),
                         g=64, block_rows=BLOCK_ROWS)
    return out


# speedup vs baseline: 10.7346x; 1.0684x over previous
import functools

import jax
import jax.numpy as jnp
from jax import lax
from jax.experimental import pallas as pl
from jax.experimental.pallas import tpu as pltpu
from jax.experimental.pallas import tpu_sc as plsc

NC = 2
NS = 16
K = 80


def _sc_segment_sum(table, src, dst, zeros_acc, with_counts):
    n, d = table.shape
    e = src.shape[0]
    nw = NC * NS
    assert e % (nw * K) == 0
    e_per_w = e // nw
    t_chunks = e_per_w // K
    row_step = (n // NS) // 8 * 8
    row_span = n - row_step * (NS - 1)
    assert row_span % 8 == 0 and row_span >= row_step

    mesh = plsc.VectorSubcoreMesh(core_axis_name="c", subcore_axis_name="s")

    nbuf = 2
    out_type = [jax.ShapeDtypeStruct((NC, n, d), jnp.float32)]
    scratch = [
        pltpu.VMEM((e_per_w,), jnp.int32),
        pltpu.VMEM((t_chunks, K), jnp.int32),
        pltpu.VMEM_SHARED((n, d), jnp.float32),
    ]
    scratch += [pltpu.VMEM((K, d), jnp.float32) for _ in range(nbuf)]
    scratch += [pltpu.SemaphoreType.DMA for _ in range(2 * nbuf + 1)]
    if with_counts:
        out_type.append(jax.ShapeDtypeStruct((NC * n,), jnp.float32))
        scratch.append(pltpu.VMEM((K,), jnp.float32))
        scratch.append(pltpu.VMEM_SHARED((n,), jnp.float32))
        scratch.append(pltpu.VMEM((K,), jnp.float32))

    def body(table_h, src_h, dst_h, zacc_h, out_h, *rest):
        cntrow_v = None
        if with_counts:
            cnt_out_h = rest[0]
            rest = rest[1:]
        src_v, dst_v, acc_s = rest[0], rest[1], rest[2]
        rows = list(rest[3:3 + nbuf])
        gsem = list(rest[3 + nbuf:3 + 2 * nbuf])
        ssem = list(rest[3 + 2 * nbuf:3 + 3 * nbuf])
        csem = rest[3 + 3 * nbuf]
        rows_v = rows[0]
        if with_counts:
            ones_v, cnt_s, cntrow_v = rest[3 + 3 * nbuf + 1:]
        c = lax.axis_index("c")
        s = lax.axis_index("s")
        wid = s * NC + c
        base_e = wid * e_per_w
        r0 = s * row_step
        n_sub = row_span // K

        pltpu.sync_copy(zacc_h.at[pl.ds(0, K)], rows_v)
        if with_counts:
            for i in range(K // 16):
                ones_v[pl.ds(i * 16, 16)] = jnp.full((16,), 1.0, jnp.float32)
                cntrow_v[pl.ds(i * 16, 16)] = jnp.zeros((16,), jnp.float32)

        @pl.loop(0, n_sub)
        def _(j):
            pltpu.sync_copy(rows_v, acc_s.at[pl.ds(r0 + j * K, K)])

        if with_counts:
            @pl.loop(0, n_sub)
            def _(j):
                pltpu.sync_copy(cntrow_v, cnt_s.at[pl.ds(r0 + j * K, K)])

        pltpu.sync_copy(src_h.at[pl.ds(base_e, e_per_w)], src_v)
        pltpu.sync_copy(dst_h.at[wid], dst_v)

        plsc.subcore_barrier()

        def start_gather(t, b):
            return pltpu.async_copy(
                table_h.at[src_v.at[pl.ds(t * K, K)]], rows[b], gsem[b])

        def start_scatter(t, b):
            sd = pltpu.async_copy(rows[b], acc_s.at[dst_v.at[t]], ssem[b],
                                  add=True)
            cd = (pltpu.async_copy(ones_v, cnt_s.at[dst_v.at[t]], csem,
                                   add=True) if with_counts else None)
            return sd, cd

        def drain_scatter(b):
            pltpu.make_async_copy(rows[b], acc_s.at[dst_v.at[0]],
                                  ssem[b]).wait()
            if with_counts:
                pltpu.make_async_copy(ones_v, cnt_s.at[dst_v.at[0]],
                                      csem).wait()

        n_full = t_chunks // nbuf
        rem = t_chunks - n_full * nbuf

        @pl.loop(0, n_full)
        def _(tt):
            gd = []
            for b in range(nbuf):
                @pl.when(tt > 0)
                def _():
                    drain_scatter(b)
                gd.append(start_gather(tt * nbuf + b, b))
            for b in range(nbuf):
                gd[b].wait()
                start_scatter(tt * nbuf + b, b)

        for b in range(rem):
            drain_scatter(b)
            g = start_gather(n_full * nbuf + b, b)
            g.wait()
            start_scatter(n_full * nbuf + b, b)
        for b in range(rem, nbuf):
            drain_scatter(b)
        for b in range(rem):
            drain_scatter(b)

        plsc.subcore_barrier()

        @pl.loop(0, n_sub)
        def _(j):
            pltpu.sync_copy(acc_s.at[pl.ds(r0 + j * K, K)], rows_v)
            pltpu.sync_copy(rows_v, out_h.at[c, pl.ds(r0 + j * K, K)])

        if with_counts:
            @pl.loop(0, n_sub)
            def _(j):
                pltpu.sync_copy(cnt_s.at[pl.ds(r0 + j * K, K)], cntrow_v)
                pltpu.sync_copy(cntrow_v,
                                cnt_out_h.at[pl.ds(c * n + r0 + j * K, K)])

    fn = pl.kernel(body, out_type=tuple(out_type), mesh=mesh,
                   scratch_types=tuple(scratch))
    dst3 = dst.reshape(nw, t_chunks, K)
    return fn(table, src, dst3, zeros_acc)


def _tc_layer_body(sp_ref, cnt_ref, x_ref, wl_ref, b_ref, wr_ref, h_ref):
    ssum = sp_ref[0] + sp_ref[1]
    agg = ssum / jnp.maximum(cnt_ref[...], 1.0)
    t = (jnp.dot(agg, wl_ref[...], preferred_element_type=jnp.float32)
         + jnp.dot(x_ref[...], wr_ref[...], preferred_element_type=jnp.float32)
         + b_ref[...])
    nrm = jnp.sqrt(jnp.sum(t * t, axis=1, keepdims=True))
    t = t / jnp.maximum(nrm, 1e-12)
    h_ref[...] = jnp.maximum(t, 0.0)


def _tc_layer(sp, cnt, x, wl, b, wr, block_rows):
    n, d = x.shape
    grid = n // block_rows
    return pl.pallas_call(
        _tc_layer_body,
        grid=(grid,),
        in_specs=[
            pl.BlockSpec((NC, block_rows, d), lambda i: (0, i, 0)),
            pl.BlockSpec((block_rows, 1), lambda i: (i, 0)),
            pl.BlockSpec((block_rows, d), lambda i: (i, 0)),
            pl.BlockSpec((d, d), lambda i: (0, 0)),
            pl.BlockSpec((1, d), lambda i: (0, 0)),
            pl.BlockSpec((d, d), lambda i: (0, 0)),
        ],
        out_specs=pl.BlockSpec((block_rows, d), lambda i: (i, 0)),
        out_shape=jax.ShapeDtypeStruct((n, d), jnp.float32),
    )(sp, cnt, x, wl, b, wr)


def _tc_layer_pool_body(g, sp_ref, cnt_ref, x_ref, wl_ref, b_ref, wr_ref,
                        batch_ref, wlin_ref, blin_ref, out_ref,
                        gsum_ref, gcnt_ref):
    i = pl.program_id(0)
    ssum = sp_ref[0] + sp_ref[1]
    agg = ssum / jnp.maximum(cnt_ref[...], 1.0)
    t = (jnp.dot(agg, wl_ref[...], preferred_element_type=jnp.float32)
         + jnp.dot(x_ref[...], wr_ref[...], preferred_element_type=jnp.float32)
         + b_ref[...])
    nrm = jnp.sqrt(jnp.sum(t * t, axis=1, keepdims=True))
    h = jnp.maximum(t / jnp.maximum(nrm, 1e-12), 0.0)

    @pl.when(i == 0)
    def _():
        gsum_ref[...] = jnp.zeros_like(gsum_ref)
        gcnt_ref[...] = jnp.zeros_like(gcnt_ref)

    bids = batch_ref[0]
    gids = lax.broadcasted_iota(jnp.int32, (gsum_ref.shape[0], h.shape[0]), 0)
    m = (gids == bids).astype(jnp.float32)
    gsum_ref[...] += jnp.dot(m, h, preferred_element_type=jnp.float32)
    gcnt_ref[...] += jnp.sum(m, axis=1, keepdims=True)

    @pl.when(i == g - 1)
    def _():
        gmean = gsum_ref[...] / jnp.maximum(gcnt_ref[...], 1.0)
        out_ref[...] = (jnp.dot(gmean, wlin_ref[...],
                                preferred_element_type=jnp.float32)
                        + blin_ref[...])


def _tc_layer_pool(sp, cnt, x, wl, b, wr, batch3d, wlin, blin, g, block_rows):
    n, d = x.shape
    o = wlin.shape[1]
    grid = n // block_rows
    return pl.pallas_call(
        functools.partial(_tc_layer_pool_body, grid),
        grid=(grid,),
        in_specs=[
            pl.BlockSpec((NC, block_rows, d), lambda i: (0, i, 0)),
            pl.BlockSpec((block_rows, 1), lambda i: (i, 0)),
            pl.BlockSpec((block_rows, d), lambda i: (i, 0)),
            pl.BlockSpec((d, d), lambda i: (0, 0)),
            pl.BlockSpec((1, d), lambda i: (0, 0)),
            pl.BlockSpec((d, d), lambda i: (0, 0)),
            pl.BlockSpec((1, 1, block_rows), lambda i: (i, 0, 0)),
            pl.BlockSpec((d, o), lambda i: (0, 0)),
            pl.BlockSpec((1, o), lambda i: (0, 0)),
        ],
        out_specs=pl.BlockSpec((g, o), lambda i: (0, 0)),
        out_shape=jax.ShapeDtypeStruct((g, o), jnp.float32),
        scratch_shapes=[
            pltpu.VMEM((g, d), jnp.float32),
            pltpu.VMEM((g, 1), jnp.float32),
        ],
    )(sp, cnt, x, wl, b, wr, batch3d, wlin, blin)


BLOCK_ROWS = 1000


def kernel(x, edge_index, batch, W1l, b1, W1r, W2l, b2, W2r, Wlin, blin):
    n, d = x.shape
    src = edge_index[0]
    dst = edge_index[1]
    zeros_acc = jnp.zeros((n, d), jnp.float32)

    s1p, cntflat = _sc_segment_sum(x, src, dst, zeros_acc, with_counts=True)
    cnt = (cntflat[:n] + cntflat[n:]).reshape(n, 1)
    h1 = _tc_layer(s1p, cnt, x, W1l, b1.reshape(1, -1), W1r, BLOCK_ROWS)
    (s2p,) = _sc_segment_sum(h1, src, dst, zeros_acc, with_counts=False)
    batch3d = batch.reshape(n // BLOCK_ROWS, 1, BLOCK_ROWS)
    out = _tc_layer_pool(s2p, cnt, h1, W2l, b2.reshape(1, -1), W2r,
                         batch3d, Wlin, blin.reshape(1, -1),
                         g=64, block_rows=BLOCK_ROWS)
    return out


# speedup vs baseline: 12.2993x; 1.1458x over previous
import functools

import jax
import jax.numpy as jnp
from jax import lax
from jax.experimental import pallas as pl
from jax.experimental.pallas import tpu as pltpu
from jax.experimental.pallas import tpu_sc as plsc

NC = 2
NS = 16
K = 80


def _sc_segment_sum(table, src, dst, zeros_acc, with_counts):
    n, d = table.shape
    e = src.shape[0]
    nw = NC * NS
    assert e % (nw * K) == 0
    e_per_w = e // nw
    t_chunks = e_per_w // K
    row_step = (n // NS) // 8 * 8
    row_span = n - row_step * (NS - 1)
    assert row_span % 8 == 0 and row_span >= row_step

    mesh = plsc.VectorSubcoreMesh(core_axis_name="c", subcore_axis_name="s")

    nbuf = 4
    n_pass = 5
    assert t_chunks % n_pass == 0
    cp = t_chunks // n_pass
    pass_e = cp * K
    out_type = [jax.ShapeDtypeStruct((NC, n, d), jnp.float32)]
    scratch = [
        pltpu.VMEM((pass_e,), jnp.int32),
        pltpu.VMEM((cp, K), jnp.int32),
        pltpu.VMEM_SHARED((n, d), jnp.float32),
    ]
    scratch += [pltpu.VMEM((K, d), jnp.float32) for _ in range(nbuf)]
    scratch += [pltpu.SemaphoreType.DMA for _ in range(2 * nbuf + 1)]
    if with_counts:
        out_type.append(jax.ShapeDtypeStruct((NC * n,), jnp.float32))
        scratch.append(pltpu.VMEM((K,), jnp.float32))
        scratch.append(pltpu.VMEM_SHARED((n,), jnp.float32))
        scratch.append(pltpu.VMEM((K,), jnp.float32))

    def body(table_h, src_h, dst_h, zacc_h, out_h, *rest):
        cntrow_v = None
        if with_counts:
            cnt_out_h = rest[0]
            rest = rest[1:]
        src_v, dst_v, acc_s = rest[0], rest[1], rest[2]
        rows = list(rest[3:3 + nbuf])
        gsem = list(rest[3 + nbuf:3 + 2 * nbuf])
        ssem = list(rest[3 + 2 * nbuf:3 + 3 * nbuf])
        csem = rest[3 + 3 * nbuf]
        rows_v = rows[0]
        if with_counts:
            ones_v, cnt_s, cntrow_v = rest[3 + 3 * nbuf + 1:]
        c = lax.axis_index("c")
        s = lax.axis_index("s")
        wid = s * NC + c
        base_e = wid * e_per_w
        r0 = s * row_step
        n_sub = row_span // K

        pltpu.sync_copy(zacc_h.at[pl.ds(0, K)], rows_v)
        if with_counts:
            for i in range(K // 16):
                ones_v[pl.ds(i * 16, 16)] = jnp.full((16,), 1.0, jnp.float32)
                cntrow_v[pl.ds(i * 16, 16)] = jnp.zeros((16,), jnp.float32)

        @pl.loop(0, n_sub)
        def _(j):
            pltpu.sync_copy(rows_v, acc_s.at[pl.ds(r0 + j * K, K)])

        if with_counts:
            @pl.loop(0, n_sub)
            def _(j):
                pltpu.sync_copy(cntrow_v, cnt_s.at[pl.ds(r0 + j * K, K)])

        plsc.subcore_barrier()

        def start_gather(t, b):
            return pltpu.async_copy(
                table_h.at[src_v.at[pl.ds(t * K, K)]], rows[b], gsem[b])

        def start_scatter(t, b):
            sd = pltpu.async_copy(rows[b], acc_s.at[dst_v.at[t]], ssem[b],
                                  add=True)
            cd = (pltpu.async_copy(ones_v, cnt_s.at[dst_v.at[t]], csem,
                                   add=True) if with_counts else None)
            return sd, cd

        def drain_scatter(b):
            pltpu.make_async_copy(rows[b], acc_s.at[dst_v.at[0]],
                                  ssem[b]).wait()
            if with_counts:
                pltpu.make_async_copy(ones_v, cnt_s.at[dst_v.at[0]],
                                      csem).wait()

        n_full = cp // nbuf
        rem = cp - n_full * nbuf

        @pl.loop(0, n_pass)
        def _(p):
            pltpu.sync_copy(src_h.at[pl.ds(base_e + p * pass_e, pass_e)],
                            src_v)
            pltpu.sync_copy(dst_h.at[wid, p], dst_v)

            @pl.loop(0, n_full)
            def _(tt):
                gd = []
                for b in range(nbuf):
                    @pl.when(tt > 0)
                    def _():
                        drain_scatter(b)
                    gd.append(start_gather(tt * nbuf + b, b))
                for b in range(nbuf):
                    gd[b].wait()
                    start_scatter(tt * nbuf + b, b)

            for b in range(rem):
                drain_scatter(b)
                g = start_gather(n_full * nbuf + b, b)
                g.wait()
                start_scatter(n_full * nbuf + b, b)
            for b in range(rem, nbuf):
                drain_scatter(b)
            for b in range(rem):
                drain_scatter(b)

        plsc.subcore_barrier()

        @pl.loop(0, n_sub)
        def _(j):
            pltpu.sync_copy(acc_s.at[pl.ds(r0 + j * K, K)], rows_v)
            pltpu.sync_copy(rows_v, out_h.at[c, pl.ds(r0 + j * K, K)])

        if with_counts:
            @pl.loop(0, n_sub)
            def _(j):
                pltpu.sync_copy(cnt_s.at[pl.ds(r0 + j * K, K)], cntrow_v)
                pltpu.sync_copy(cntrow_v,
                                cnt_out_h.at[pl.ds(c * n + r0 + j * K, K)])

    fn = pl.kernel(body, out_type=tuple(out_type), mesh=mesh,
                   scratch_types=tuple(scratch))
    dst4 = dst.reshape(nw, n_pass, cp, K)
    return fn(table, src, dst4, zeros_acc)


def _tc_layer_body(sp_ref, cnt_ref, x_ref, wl_ref, b_ref, wr_ref, h_ref):
    ssum = sp_ref[0] + sp_ref[1]
    agg = ssum / jnp.maximum(cnt_ref[...], 1.0)
    t = (jnp.dot(agg, wl_ref[...], preferred_element_type=jnp.float32)
         + jnp.dot(x_ref[...], wr_ref[...], preferred_element_type=jnp.float32)
         + b_ref[...])
    nrm = jnp.sqrt(jnp.sum(t * t, axis=1, keepdims=True))
    t = t / jnp.maximum(nrm, 1e-12)
    h_ref[...] = jnp.maximum(t, 0.0)


def _tc_layer(sp, cnt, x, wl, b, wr, block_rows):
    n, d = x.shape
    grid = n // block_rows
    return pl.pallas_call(
        _tc_layer_body,
        grid=(grid,),
        in_specs=[
            pl.BlockSpec((NC, block_rows, d), lambda i: (0, i, 0)),
            pl.BlockSpec((block_rows, 1), lambda i: (i, 0)),
            pl.BlockSpec((block_rows, d), lambda i: (i, 0)),
            pl.BlockSpec((d, d), lambda i: (0, 0)),
            pl.BlockSpec((1, d), lambda i: (0, 0)),
            pl.BlockSpec((d, d), lambda i: (0, 0)),
        ],
        out_specs=pl.BlockSpec((block_rows, d), lambda i: (i, 0)),
        out_shape=jax.ShapeDtypeStruct((n, d), jnp.float32),
    )(sp, cnt, x, wl, b, wr)


def _tc_layer_pool_body(g, sp_ref, cnt_ref, x_ref, wl_ref, b_ref, wr_ref,
                        batch_ref, wlin_ref, blin_ref, out_ref,
                        gsum_ref, gcnt_ref):
    i = pl.program_id(0)
    ssum = sp_ref[0] + sp_ref[1]
    agg = ssum / jnp.maximum(cnt_ref[...], 1.0)
    t = (jnp.dot(agg, wl_ref[...], preferred_element_type=jnp.float32)
         + jnp.dot(x_ref[...], wr_ref[...], preferred_element_type=jnp.float32)
         + b_ref[...])
    nrm = jnp.sqrt(jnp.sum(t * t, axis=1, keepdims=True))
    h = jnp.maximum(t / jnp.maximum(nrm, 1e-12), 0.0)

    @pl.when(i == 0)
    def _():
        gsum_ref[...] = jnp.zeros_like(gsum_ref)
        gcnt_ref[...] = jnp.zeros_like(gcnt_ref)

    bids = batch_ref[0]
    gids = lax.broadcasted_iota(jnp.int32, (gsum_ref.shape[0], h.shape[0]), 0)
    m = (gids == bids).astype(jnp.float32)
    gsum_ref[...] += jnp.dot(m, h, preferred_element_type=jnp.float32)
    gcnt_ref[...] += jnp.sum(m, axis=1, keepdims=True)

    @pl.when(i == g - 1)
    def _():
        gmean = gsum_ref[...] / jnp.maximum(gcnt_ref[...], 1.0)
        out_ref[...] = (jnp.dot(gmean, wlin_ref[...],
                                preferred_element_type=jnp.float32)
                        + blin_ref[...])


def _tc_layer_pool(sp, cnt, x, wl, b, wr, batch3d, wlin, blin, g, block_rows):
    n, d = x.shape
    o = wlin.shape[1]
    grid = n // block_rows
    return pl.pallas_call(
        functools.partial(_tc_layer_pool_body, grid),
        grid=(grid,),
        in_specs=[
            pl.BlockSpec((NC, block_rows, d), lambda i: (0, i, 0)),
            pl.BlockSpec((block_rows, 1), lambda i: (i, 0)),
            pl.BlockSpec((block_rows, d), lambda i: (i, 0)),
            pl.BlockSpec((d, d), lambda i: (0, 0)),
            pl.BlockSpec((1, d), lambda i: (0, 0)),
            pl.BlockSpec((d, d), lambda i: (0, 0)),
            pl.BlockSpec((1, 1, block_rows), lambda i: (i, 0, 0)),
            pl.BlockSpec((d, o), lambda i: (0, 0)),
            pl.BlockSpec((1, o), lambda i: (0, 0)),
        ],
        out_specs=pl.BlockSpec((g, o), lambda i: (0, 0)),
        out_shape=jax.ShapeDtypeStruct((g, o), jnp.float32),
        scratch_shapes=[
            pltpu.VMEM((g, d), jnp.float32),
            pltpu.VMEM((g, 1), jnp.float32),
        ],
    )(sp, cnt, x, wl, b, wr, batch3d, wlin, blin)


BLOCK_ROWS = 1000


def kernel(x, edge_index, batch, W1l, b1, W1r, W2l, b2, W2r, Wlin, blin):
    n, d = x.shape
    src = edge_index[0]
    dst = edge_index[1]
    zeros_acc = jnp.zeros((n, d), jnp.float32)

    s1p, cntflat = _sc_segment_sum(x, src, dst, zeros_acc, with_counts=True)
    cnt = (cntflat[:n] + cntflat[n:]).reshape(n, 1)
    h1 = _tc_layer(s1p, cnt, x, W1l, b1.reshape(1, -1), W1r, BLOCK_ROWS)
    (s2p,) = _sc_segment_sum(h1, src, dst, zeros_acc, with_counts=False)
    batch3d = batch.reshape(n // BLOCK_ROWS, 1, BLOCK_ROWS)
    out = _tc_layer_pool(s2p, cnt, h1, W2l, b2.reshape(1, -1), W2r,
                         batch3d, Wlin, blin.reshape(1, -1),
                         g=64, block_rows=BLOCK_ROWS)
    return out
